# Initial kernel scaffold; baseline (speedup 1.0000x reference)
import jax
import jax.numpy as jnp
from jax.experimental import pallas as pl


def kernel(x, weight):
    raise NotImplementedError("write your pallas kernel here")


# speedup vs baseline: 2.8997x; 2.8997x over previous
import functools

import jax
import jax.numpy as jnp
from jax import lax
from jax.experimental import pallas as pl
from jax.experimental.pallas import tpu as pltpu
from jax.experimental.pallas import tpu_sc as plsc

L = 20
D = 64
CB = 128


def _make_sc_call(num_bags, nw):
    chunks_per_w = num_bags // (nw * CB)
    bags_per_w = chunks_per_w * CB
    mesh = plsc.VectorSubcoreMesh(core_axis_name="c", subcore_axis_name="s")
    nc = mesh.num_cores

    @functools.partial(
        pl.kernel,
        out_type=jax.ShapeDtypeStruct((num_bags, D), jnp.float32),
        mesh=mesh,
        compiler_params=pltpu.CompilerParams(use_tc_tiling_on_sc=False),
        scratch_types=[
            pltpu.VMEM((L, chunks_per_w, CB), jnp.int32),
            pltpu.VMEM((CB, D), jnp.float32),
            pltpu.VMEM((CB, D), jnp.float32),
            pltpu.SemaphoreType.DMA,
            pltpu.SemaphoreType.DMA,
            pltpu.SemaphoreType.DMA,
        ],
    )
    def sc_call(xq_hbm, w_hbm, out_hbm, idx_v, acc_v, outb_v, sem_i, sem_g, sem_o):
        wid = lax.axis_index("s") * nc + lax.axis_index("c")
        bag0 = wid * bags_per_w

        for j in range(L):
            pltpu.async_copy(xq_hbm.at[j, wid], idx_v.at[j], sem_i)
        for j in range(L):
            pltpu.make_async_copy(xq_hbm.at[j, wid], idx_v.at[j], sem_i).wait()

        def chunk_body(lc, _):
            pltpu.async_copy(w_hbm.at[idx_v.at[0, lc]], acc_v, sem_g).wait()
            for j in range(1, L):
                pltpu.async_copy(w_hbm.at[idx_v.at[j, lc]], acc_v, sem_g,
                                 add=True)
            for j in range(1, L):
                pltpu.make_async_copy(w_hbm.at[idx_v.at[j, lc]], acc_v,
                                      sem_g).wait()
            def scale_body(b, _):
                for k in range(D // 16):
                    outb_v[b, pl.ds(16 * k, 16)] = (
                        acc_v[b, pl.ds(16 * k, 16)] * (1.0 / L))
                return 0
            lax.fori_loop(0, CB, scale_body, 0, unroll=2)
            copy_o = pltpu.async_copy(
                outb_v, out_hbm.at[pl.ds(bag0 + lc * CB, CB)], sem_o)
            copy_o.wait()
            return 0

        lax.fori_loop(0, chunks_per_w, chunk_body, 0)

    return sc_call


def kernel(x, weight):
    b, f, l = x.shape
    num_bags = b * f
    info = plsc.get_sparse_core_info()
    nw = info.num_cores * info.num_subcores
    xt = x.reshape(num_bags, l).T
    xq = xt.reshape(l, nw, num_bags // (nw * CB), CB)
    out = _make_sc_call(num_bags, nw)(xq, weight)
    return out.reshape(b, f, D)


# speedup vs baseline: 3.1442x; 1.0843x over previous
import functools

import jax
import jax.numpy as jnp
from jax import lax
from jax.experimental import pallas as pl
from jax.experimental.pallas import tpu as pltpu
from jax.experimental.pallas import tpu_sc as plsc

L = 20
D = 64
CB = 128


def _make_sc_call(num_bags, nw):
    chunks_per_w = num_bags // (nw * CB)
    bags_per_w = chunks_per_w * CB
    mesh = plsc.VectorSubcoreMesh(core_axis_name="c", subcore_axis_name="s")
    nc = mesh.num_cores

    @functools.partial(
        pl.kernel,
        out_type=jax.ShapeDtypeStruct((num_bags, D), jnp.float32),
        mesh=mesh,
        compiler_params=pltpu.CompilerParams(use_tc_tiling_on_sc=False),
        scratch_types=[
            pltpu.VMEM((L, chunks_per_w, CB), jnp.int32),
            pltpu.VMEM((2, CB, D), jnp.float32),
            pltpu.VMEM((2, CB, D), jnp.float32),
            pltpu.SemaphoreType.DMA,
            pltpu.SemaphoreType.DMA((2,)),
            pltpu.SemaphoreType.DMA((2,)),
        ],
    )
    def sc_call(xq_hbm, w_hbm, out_hbm, idx_v, acc_v, outb_v, sem_i, sem_g, sem_o):
        wid = lax.axis_index("s") * nc + lax.axis_index("c")
        bag0 = wid * bags_per_w
        zeros16 = jnp.zeros((16,), jnp.float32)

        for j in range(L):
            pltpu.async_copy(xq_hbm.at[j, wid], idx_v.at[j], sem_i)

        def zero_body(b, _):
            for s in range(2):
                for k in range(D // 16):
                    acc_v[s, b, pl.ds(16 * k, 16)] = zeros16
            return 0
        lax.fori_loop(0, CB, zero_body, 0, unroll=4)

        for j in range(L):
            pltpu.make_async_copy(xq_hbm.at[j, wid], idx_v.at[j], sem_i).wait()

        def chunk_body(c, _):
            slot = lax.rem(c, 2)
            oslot = 1 - slot

            @pl.when(c < chunks_per_w)
            def _fire():
                for j in range(L):
                    pltpu.async_copy(w_hbm.at[idx_v.at[j, c]], acc_v.at[slot],
                                     sem_g.at[slot], add=True)

            @pl.when(c >= 1)
            def _retire():
                p = c - 1
                for j in range(L):
                    pltpu.make_async_copy(w_hbm.at[idx_v.at[j, p]],
                                          acc_v.at[oslot],
                                          sem_g.at[oslot]).wait()
                @pl.when(c >= 3)
                def _wait_out():
                    pltpu.make_async_copy(
                        outb_v.at[oslot],
                        out_hbm.at[pl.ds(bag0 + (p - 2) * CB, CB)],
                        sem_o.at[oslot]).wait()

                def scale_body(b, _):
                    for k in range(D // 16):
                        sl = pl.ds(16 * k, 16)
                        outb_v[oslot, b, sl] = acc_v[oslot, b, sl] * (1.0 / L)
                        acc_v[oslot, b, sl] = zeros16
                    return 0
                lax.fori_loop(0, CB, scale_body, 0, unroll=4)

                pltpu.async_copy(outb_v.at[oslot],
                                 out_hbm.at[pl.ds(bag0 + p * CB, CB)],
                                 sem_o.at[oslot])
            return 0

        lax.fori_loop(0, chunks_per_w + 1, chunk_body, 0)

        for p in (chunks_per_w - 2, chunks_per_w - 1):
            s = p % 2
            pltpu.make_async_copy(outb_v.at[s],
                                  out_hbm.at[pl.ds(bag0 + p * CB, CB)],
                                  sem_o.at[s]).wait()

    return sc_call


def kernel(x, weight):
    b, f, l = x.shape
    num_bags = b * f
    info = plsc.get_sparse_core_info()
    nw = info.num_cores * info.num_subcores
    xt = x.reshape(num_bags, l).T
    xq = xt.reshape(l, nw, num_bags // (nw * CB), CB)
    out = _make_sc_call(num_bags, nw)(xq, weight)
    return out.reshape(b, f, D)
